# baseline (speedup 1.0000x reference)
import functools

import jax
import jax.numpy as jnp
from jax import lax
from jax.experimental import pallas as pl
from jax.experimental.pallas import tpu as pltpu
from jax.experimental.pallas import tpu_sc as plsc

_INTERPRET = False
_USE_SC = True

E = 8
D = 1024
F = 1024
N = 2048
DP = D // 2
BM = 128
NBLK = 40
P = NBLK * BM
NC, NS = 2, 16
NW = NC * NS


def _router_body(x_ref, gw_ref, lb_ref, i1_ref, i2_ref, w1_ref, w2_ref):
    xb = x_ref[...]
    logits = lax.dot_general(xb, gw_ref[...], (((1,), (1,)), ((), ())),
                             preferred_element_type=jnp.float32)
    sel = logits + lb_ref[...]
    iota = lax.broadcasted_iota(jnp.int32, sel.shape, 1)
    neg = jnp.float32(-1e30)

    m1 = jnp.max(sel, axis=1, keepdims=True)
    idx1 = jnp.min(jnp.where(sel >= m1, iota, E), axis=1, keepdims=True)
    pick1 = iota == idx1
    s1 = jnp.sum(jnp.where(pick1, logits, 0.0), axis=1, keepdims=True)

    sel2 = jnp.where(pick1, neg, sel)
    m2 = jnp.max(sel2, axis=1, keepdims=True)
    idx2 = jnp.min(jnp.where(sel2 >= m2, iota, E), axis=1, keepdims=True)
    pick2 = iota == idx2
    s2 = jnp.sum(jnp.where(pick2, logits, 0.0), axis=1, keepdims=True)

    g1 = 1.0 / (1.0 + jnp.exp(-s1))
    g2 = 1.0 / (1.0 + jnp.exp(-s2))
    denom = g1 + g2 + 1e-6
    i1_ref[...] = idx1
    i2_ref[...] = idx2
    w1_ref[...] = g1 / denom
    w2_ref[...] = g2 / denom


def _router(x2d, gate_w, lb2d):
    full = lambda: pl.BlockSpec((N, 1), lambda: (0, 0))
    return pl.pallas_call(
        _router_body,
        in_specs=[
            pl.BlockSpec((N, D), lambda: (0, 0)),
            pl.BlockSpec((E, D), lambda: (0, 0)),
            pl.BlockSpec((1, E), lambda: (0, 0)),
        ],
        out_specs=[full(), full(), full(), full()],
        out_shape=[
            jax.ShapeDtypeStruct((N, 1), jnp.int32),
            jax.ShapeDtypeStruct((N, 1), jnp.int32),
            jax.ShapeDtypeStruct((N, 1), jnp.float32),
            jax.ShapeDtypeStruct((N, 1), jnp.float32),
        ],
        interpret=_INTERPRET,
    )(x2d, gate_w, lb2d)


def _metadata(i1, i2):
    flat_e = jnp.concatenate([i1, i2], axis=1).reshape(-1)
    oh = (flat_e[:, None] == jnp.arange(E, dtype=jnp.int32)[None, :]).astype(jnp.int32)
    cum = jnp.cumsum(oh, axis=0)
    counts = cum[-1]
    nblk_e = (counts + BM - 1) // BM
    blk_end = jnp.cumsum(nblk_e)
    blk_start = blk_end - nblk_e
    base_e = blk_start * BM
    rank = jnp.sum(cum * oh, axis=1) - 1
    pos = rank + jnp.take(base_e, flat_e)
    tok = jnp.arange(2 * N, dtype=jnp.int32) // 2
    slot_token = jnp.zeros((P,), jnp.int32).at[pos].set(tok)
    g_ids = jnp.arange(NBLK, dtype=jnp.int32)
    block_expert = jnp.minimum(
        jnp.sum((g_ids[:, None] >= blk_end[None, :]).astype(jnp.int32), axis=1), E - 1)
    pos2 = pos.reshape(N, 2)
    return slot_token, block_expert, pos2[:, 0], pos2[:, 1]


def _sc_gather(table, idx, n_rows, width):
    rows_per_w = n_rows // NW
    ch = rows_per_w
    while ch * width * 4 > 220 * 1024:
        ch //= 2
    n_ch = rows_per_w // ch
    mesh = plsc.VectorSubcoreMesh(core_axis_name="c", subcore_axis_name="s",
                                  num_cores=NC, num_subcores=NS)

    @functools.partial(
        pl.kernel,
        out_type=jax.ShapeDtypeStruct((n_rows, width), jnp.float32),
        mesh=mesh,
        scratch_types=[
            pltpu.VMEM((rows_per_w,), jnp.int32),
            pltpu.VMEM((ch, width), jnp.float32),
            pltpu.VMEM((ch, width), jnp.float32),
            pltpu.SemaphoreType.DMA,
            pltpu.SemaphoreType.DMA,
        ],
    )
    def k(table_hbm, idx_hbm, out_hbm, idx_v, buf0, buf1, sem0, sem1):
        wid = lax.axis_index("s") * NC + lax.axis_index("c")
        base = wid * rows_per_w
        pltpu.sync_copy(idx_hbm.at[pl.ds(base, rows_per_w)], idx_v)
        bufs = (buf0, buf1)
        sems = (sem0, sem1)
        descs = [None, None]
        for c in range(n_ch):
            descs[c % 2] = pltpu.async_copy(
                table_hbm.at[idx_v.at[pl.ds(c * ch, ch)]], bufs[c % 2], sems[c % 2])
            if c > 0:
                descs[(c - 1) % 2].wait()
                pltpu.sync_copy(bufs[(c - 1) % 2],
                                out_hbm.at[pl.ds(base + (c - 1) * ch, ch)])
        descs[(n_ch - 1) % 2].wait()
        pltpu.sync_copy(bufs[(n_ch - 1) % 2],
                        out_hbm.at[pl.ds(base + (n_ch - 1) * ch, ch)])

    return k(table, idx)


def _pack(a16):
    r, d = a16.shape
    return lax.bitcast_convert_type(a16.reshape(r, d // 2, 2), jnp.float32)


def _unpack(ap):
    r, dp = ap.shape
    return lax.bitcast_convert_type(ap, jnp.bfloat16).reshape(r, dp * 2)


def _ffn_body(be_ref, xg_ref, w1_ref, w2_ref, y_ref):
    xb = xg_ref[...].astype(jnp.float32)
    h = lax.dot_general(xb, w1_ref[0], (((1,), (1,)), ((), ())),
                        preferred_element_type=jnp.float32)
    h = jnp.square(jnp.maximum(h, 0.0))
    y_ref[...] = lax.dot_general(h, w2_ref[0], (((1,), (1,)), ((), ())),
                                 preferred_element_type=jnp.float32
                                 ).astype(jnp.bfloat16)


def _ffn(xg16, w1, w2, block_expert):
    grid_spec = pltpu.PrefetchScalarGridSpec(
        num_scalar_prefetch=1,
        grid=(NBLK,),
        in_specs=[
            pl.BlockSpec((BM, D), lambda g, be: (g, 0)),
            pl.BlockSpec((1, F, D), lambda g, be: (be[g], 0, 0)),
            pl.BlockSpec((1, D, F), lambda g, be: (be[g], 0, 0)),
        ],
        out_specs=pl.BlockSpec((BM, D), lambda g, be: (g, 0)),
    )
    return pl.pallas_call(
        _ffn_body,
        grid_spec=grid_spec,
        out_shape=jax.ShapeDtypeStruct((P, D), jnp.bfloat16),
        interpret=_INTERPRET,
    )(block_expert, xg16, w1, w2)


_SB = 256


def _shared_body(x_ref, sfc_ref, spr_ref, out_ref):
    hs = lax.dot_general(x_ref[...], sfc_ref[...], (((1,), (1,)), ((), ())),
                         preferred_element_type=jnp.float32)
    hs = jnp.square(jnp.maximum(hs, 0.0))
    out_ref[...] = lax.dot_general(hs, spr_ref[...], (((1,), (1,)), ((), ())),
                                   preferred_element_type=jnp.float32)


def _shared(x2d, sfc, spr):
    return pl.pallas_call(
        _shared_body,
        grid=(N // _SB,),
        in_specs=[
            pl.BlockSpec((_SB, D), lambda g: (g, 0)),
            pl.BlockSpec((F, D), lambda g: (0, 0)),
            pl.BlockSpec((D, F), lambda g: (0, 0)),
        ],
        out_specs=pl.BlockSpec((_SB, D), lambda g: (g, 0)),
        out_shape=jax.ShapeDtypeStruct((N, D), jnp.float32),
        interpret=_INTERPRET,
    )(x2d, sfc, spr)


_CB = 256


def _combine_body(sh_ref, a_ref, b_ref, wa_ref, wb_ref, out_ref):
    out_ref[...] = (sh_ref[...]
                    + wa_ref[...] * a_ref[...].astype(jnp.float32)
                    + wb_ref[...] * b_ref[...].astype(jnp.float32))


def _combine(sh, a16, b16, wa, wb):
    blk = lambda g: (g, 0)
    return pl.pallas_call(
        _combine_body,
        grid=(N // _CB,),
        in_specs=[
            pl.BlockSpec((_CB, D), blk),
            pl.BlockSpec((_CB, D), blk),
            pl.BlockSpec((_CB, D), blk),
            pl.BlockSpec((_CB, 1), blk),
            pl.BlockSpec((_CB, 1), blk),
        ],
        out_specs=pl.BlockSpec((_CB, D), blk),
        out_shape=jax.ShapeDtypeStruct((N, D), jnp.float32),
        interpret=_INTERPRET,
    )(sh, a16, b16, wa, wb)


def kernel(x, gate_w, lb_bias, w1, w2, shared_fc, shared_proj):
    bsz, t, d = x.shape
    x2d = x.reshape(t * bsz, d)
    i1, i2, wa, wb = _router(x2d, gate_w, lb_bias.reshape(1, E))
    slot_token, block_expert, pos0, pos1 = _metadata(i1, i2)
    sh = _shared(x2d, shared_fc, shared_proj)
    xp = _pack(x2d.astype(jnp.bfloat16))
    if _USE_SC:
        xg16 = _unpack(_sc_gather(xp, slot_token, P, DP))
    else:
        xg16 = _unpack(jnp.take(xp, slot_token, axis=0))
    y16 = _ffn(xg16, w1, w2, block_expert)
    yp = _pack(y16)
    pos_ab = jnp.concatenate([pos0, pos1])
    if _USE_SC:
        ab = _sc_gather(yp, pos_ab, 2 * N, DP)
    else:
        ab = jnp.take(yp, pos_ab, axis=0)
    a16 = _unpack(ab[:N])
    b16 = _unpack(ab[N:])
    out = _combine(sh, a16, b16, wa, wb)
    return out.reshape(bsz, t, d)


# speedup vs baseline: 3.4452x; 3.4452x over previous
import functools

import jax
import jax.numpy as jnp
from jax import lax
from jax.experimental import pallas as pl
from jax.experimental.pallas import tpu as pltpu
from jax.experimental.pallas import tpu_sc as plsc

_INTERPRET = False
_USE_SC = True
_DYN_GRID = False

E = 8
D = 1024
F = 1024
N = 2048
DP = D // 2
BM = 128
NBLK = 40
P = NBLK * BM
NC, NS = 2, 16
NW = NC * NS
TPW = N // NW


def _router_body(x_ref, gw_ref, lb_ref,
                 wa_ref, wb_ref, pos_ref, be_ref, nb_ref):
    xb = x_ref[...]
    logits = lax.dot_general(xb, gw_ref[...], (((1,), (1,)), ((), ())),
                             preferred_element_type=jnp.float32)
    sel = logits + lb_ref[...]
    iota = lax.broadcasted_iota(jnp.int32, sel.shape, 1)
    neg = jnp.float32(-1e30)

    m1 = jnp.max(sel, axis=1, keepdims=True)
    idx1 = jnp.min(jnp.where(sel >= m1, iota, E), axis=1, keepdims=True)
    pick1 = iota == idx1
    s1 = jnp.sum(jnp.where(pick1, logits, 0.0), axis=1, keepdims=True)

    sel2 = jnp.where(pick1, neg, sel)
    m2 = jnp.max(sel2, axis=1, keepdims=True)
    idx2 = jnp.min(jnp.where(sel2 >= m2, iota, E), axis=1, keepdims=True)
    pick2 = iota == idx2
    s2 = jnp.sum(jnp.where(pick2, logits, 0.0), axis=1, keepdims=True)

    g1 = 1.0 / (1.0 + jnp.exp(-s1))
    g2 = 1.0 / (1.0 + jnp.exp(-s2))
    denom = g1 + g2 + 1e-6
    wa_ref[...] = g1 / denom
    wb_ref[...] = g2 / denom

    oh = jnp.concatenate([pick1, pick2], axis=0).astype(jnp.int32)
    cum = oh
    sh = 1
    while sh < 2 * N:
        top = jnp.zeros((sh, E), jnp.int32)
        cum = cum + jnp.concatenate([top, cum[: 2 * N - sh]], axis=0)
        sh *= 2
    counts = cum[2 * N - 1 : 2 * N, :]
    nblk_e = (counts + (BM - 1)) // BM
    ii = lax.broadcasted_iota(jnp.int32, (E, E), 0)
    jj = lax.broadcasted_iota(jnp.int32, (E, E), 1)
    tri = (ii <= jj).astype(jnp.float32)
    blk_end = lax.dot_general(nblk_e.astype(jnp.float32), tri,
                              (((1,), (0,)), ((), ())),
                              preferred_element_type=jnp.float32).astype(jnp.int32)
    base_e = (blk_end - nblk_e) * BM
    rank = jnp.sum(cum * oh, axis=1, keepdims=True) - 1
    base_q = jnp.sum(oh * base_e, axis=1, keepdims=True)
    pos_ref[...] = rank + base_q

    gi = lax.broadcasted_iota(jnp.int32, (NBLK, E), 0)
    be = jnp.sum((gi >= blk_end).astype(jnp.int32), axis=1, keepdims=True)
    be_ref[...] = jnp.minimum(be, E - 1)
    nb_ref[...] = blk_end[:, E - 1 :]


def _router(x2d, gate_w, lb2d):
    col = lambda n: pl.BlockSpec((n, 1), lambda: (0, 0))
    return pl.pallas_call(
        _router_body,
        in_specs=[
            pl.BlockSpec((N, D), lambda: (0, 0)),
            pl.BlockSpec((E, D), lambda: (0, 0)),
            pl.BlockSpec((1, E), lambda: (0, 0)),
        ],
        out_specs=[col(N), col(N), col(2 * N), col(NBLK), col(1)],
        out_shape=[
            jax.ShapeDtypeStruct((N, 1), jnp.float32),
            jax.ShapeDtypeStruct((N, 1), jnp.float32),
            jax.ShapeDtypeStruct((2 * N, 1), jnp.int32),
            jax.ShapeDtypeStruct((NBLK, 1), jnp.int32),
            jax.ShapeDtypeStruct((1, 1), jnp.int32),
        ],
        interpret=_INTERPRET,
    )(x2d, gate_w, lb2d)


def _sc_dispatch(x2d, pos3):
    mesh = plsc.VectorSubcoreMesh(core_axis_name="c", subcore_axis_name="s",
                                  num_cores=NC, num_subcores=NS)

    @functools.partial(
        pl.kernel,
        out_type=jax.ShapeDtypeStruct((P, D), jnp.float32),
        mesh=mesh,
        scratch_types=[
            pltpu.VMEM((2, TPW), jnp.int32),
            pltpu.VMEM((TPW, D), jnp.float32),
            pltpu.SemaphoreType.DMA,
        ],
    )
    def k(xp_hbm, pos_hbm, out_hbm, idx_v, xbuf, sem):
        wid = lax.axis_index("s") * NC + lax.axis_index("c")
        pltpu.sync_copy(pos_hbm.at[wid], idx_v)
        pltpu.sync_copy(xp_hbm.at[pl.ds(wid * TPW, TPW)], xbuf)
        d0 = pltpu.async_copy(xbuf, out_hbm.at[idx_v.at[0]], sem)
        d1 = pltpu.async_copy(xbuf, out_hbm.at[idx_v.at[1]], sem)
        d0.wait()
        d1.wait()

    return k(x2d, pos3)


def _sc_gather(table, idx, n_rows, width):
    rows_per_w = n_rows // NW
    ch = rows_per_w
    while ch * width * 4 > 220 * 1024:
        ch //= 2
    n_ch = rows_per_w // ch
    mesh = plsc.VectorSubcoreMesh(core_axis_name="c", subcore_axis_name="s",
                                  num_cores=NC, num_subcores=NS)

    @functools.partial(
        pl.kernel,
        out_type=jax.ShapeDtypeStruct((n_rows, width), jnp.float32),
        mesh=mesh,
        scratch_types=[
            pltpu.VMEM((rows_per_w,), jnp.int32),
            pltpu.VMEM((ch, width), jnp.float32),
            pltpu.VMEM((ch, width), jnp.float32),
            pltpu.SemaphoreType.DMA,
            pltpu.SemaphoreType.DMA,
        ],
    )
    def k(table_hbm, idx_hbm, out_hbm, idx_v, buf0, buf1, sem0, sem1):
        wid = lax.axis_index("s") * NC + lax.axis_index("c")
        base = wid * rows_per_w
        pltpu.sync_copy(idx_hbm.at[pl.ds(base, rows_per_w)], idx_v)
        bufs = (buf0, buf1)
        sems = (sem0, sem1)
        descs = [None, None]
        for c in range(n_ch):
            descs[c % 2] = pltpu.async_copy(
                table_hbm.at[idx_v.at[pl.ds(c * ch, ch)]], bufs[c % 2], sems[c % 2])
            if c > 0:
                descs[(c - 1) % 2].wait()
                pltpu.sync_copy(bufs[(c - 1) % 2],
                                out_hbm.at[pl.ds(base + (c - 1) * ch, ch)])
        descs[(n_ch - 1) % 2].wait()
        pltpu.sync_copy(bufs[(n_ch - 1) % 2],
                        out_hbm.at[pl.ds(base + (n_ch - 1) * ch, ch)])

    return k(table, idx)


def _pack(a16):
    r, d = a16.shape
    return lax.bitcast_convert_type(a16.reshape(r, d // 2, 2), jnp.float32)


def _unpack(ap):
    r, dp = ap.shape
    return lax.bitcast_convert_type(ap, jnp.bfloat16).reshape(r, dp * 2)


def _ffn_body(be_ref, xg_ref, w1_ref, w2_ref, y_ref):
    xb = xg_ref[...]
    h = lax.dot_general(xb, w1_ref[0], (((1,), (1,)), ((), ())),
                        preferred_element_type=jnp.float32)
    h = jnp.square(jnp.maximum(h, 0.0))
    y_ref[...] = lax.dot_general(h, w2_ref[0], (((1,), (1,)), ((), ())),
                                 preferred_element_type=jnp.float32)


def _ffn(xg, w1, w2, block_expert, nblk):
    grid = (nblk,) if _DYN_GRID else (NBLK,)
    grid_spec = pltpu.PrefetchScalarGridSpec(
        num_scalar_prefetch=1,
        grid=grid,
        in_specs=[
            pl.BlockSpec((BM, D), lambda g, be: (g, 0)),
            pl.BlockSpec((1, F, D), lambda g, be: (be[g], 0, 0)),
            pl.BlockSpec((1, D, F), lambda g, be: (be[g], 0, 0)),
        ],
        out_specs=pl.BlockSpec((BM, D), lambda g, be: (g, 0)),
    )
    return pl.pallas_call(
        _ffn_body,
        grid_spec=grid_spec,
        out_shape=jax.ShapeDtypeStruct((P, D), jnp.float32),
        interpret=_INTERPRET,
    )(block_expert, xg, w1, w2)


_SB = 256


def _shared_body(x_ref, sfc_ref, spr_ref, out_ref):
    hs = lax.dot_general(x_ref[...], sfc_ref[...], (((1,), (1,)), ((), ())),
                         preferred_element_type=jnp.float32)
    hs = jnp.square(jnp.maximum(hs, 0.0))
    out_ref[...] = lax.dot_general(hs, spr_ref[...], (((1,), (1,)), ((), ())),
                                   preferred_element_type=jnp.float32)


def _shared(x2d, sfc, spr):
    return pl.pallas_call(
        _shared_body,
        grid=(N // _SB,),
        in_specs=[
            pl.BlockSpec((_SB, D), lambda g: (g, 0)),
            pl.BlockSpec((F, D), lambda g: (0, 0)),
            pl.BlockSpec((D, F), lambda g: (0, 0)),
        ],
        out_specs=pl.BlockSpec((_SB, D), lambda g: (g, 0)),
        out_shape=jax.ShapeDtypeStruct((N, D), jnp.float32),
        interpret=_INTERPRET,
    )(x2d, sfc, spr)


_CB = 256


def _combine_body(sh_ref, a_ref, b_ref, wa_ref, wb_ref, out_ref):
    out_ref[...] = (sh_ref[...]
                    + wa_ref[...] * a_ref[...]
                    + wb_ref[...] * b_ref[...])


def _combine(sh, a, b, wa, wb):
    blk = lambda g: (g, 0)
    return pl.pallas_call(
        _combine_body,
        grid=(N // _CB,),
        in_specs=[
            pl.BlockSpec((_CB, D), blk),
            pl.BlockSpec((_CB, D), blk),
            pl.BlockSpec((_CB, D), blk),
            pl.BlockSpec((_CB, 1), blk),
            pl.BlockSpec((_CB, 1), blk),
        ],
        out_specs=pl.BlockSpec((_CB, D), blk),
        out_shape=jax.ShapeDtypeStruct((N, D), jnp.float32),
        interpret=_INTERPRET,
    )(sh, a, b, wa, wb)


def kernel(x, gate_w, lb_bias, w1, w2, shared_fc, shared_proj):
    bsz, t, d = x.shape
    x2d = x.reshape(t * bsz, d)
    wa, wb, pos, be_col, nb = _router(x2d, gate_w, lb_bias.reshape(1, E))
    pos_flat = pos.reshape(2 * N)
    block_expert = be_col.reshape(NBLK)
    sh = _shared(x2d, shared_fc, shared_proj)
    pos3 = pos_flat.reshape(2, NW, TPW).transpose(1, 0, 2)
    if _USE_SC:
        xg = _sc_dispatch(x2d, pos3)
    else:
        slot_token = jnp.zeros((P,), jnp.int32).at[pos_flat].set(
            jnp.tile(jnp.arange(N, dtype=jnp.int32), 2))
        xg = jnp.take(x2d, slot_token, axis=0)
    y = _ffn(xg, w1, w2, block_expert, nb.reshape(())[()])
    if _USE_SC:
        ab = _sc_gather(y, pos_flat, 2 * N, D)
    else:
        ab = jnp.take(y, pos_flat, axis=0)
    out = _combine(sh, ab[:N], ab[N:], wa, wb)
    return out.reshape(bsz, t, d)


# speedup vs baseline: 3.8953x; 1.1306x over previous
import functools

import jax
import jax.numpy as jnp
from jax import lax
from jax.experimental import pallas as pl
from jax.experimental.pallas import tpu as pltpu
from jax.experimental.pallas import tpu_sc as plsc

_INTERPRET = False
_USE_SC = True
_DYN_GRID = True

E = 8
D = 1024
F = 1024
N = 2048
DP = D // 2
BM = 128
NBLK = 40
P = NBLK * BM
NC, NS = 2, 16
NW = NC * NS
TPW = N // NW


def _router_body(x_ref, gw_ref, lb_ref,
                 wa_ref, wb_ref, pos_ref, be_ref, nb_ref):
    xb = x_ref[...]
    logits = lax.dot_general(xb, gw_ref[...], (((1,), (1,)), ((), ())),
                             preferred_element_type=jnp.float32)
    sel = logits + lb_ref[...]
    iota = lax.broadcasted_iota(jnp.int32, sel.shape, 1)
    neg = jnp.float32(-1e30)

    m1 = jnp.max(sel, axis=1, keepdims=True)
    idx1 = jnp.min(jnp.where(sel >= m1, iota, E), axis=1, keepdims=True)
    pick1 = iota == idx1
    s1 = jnp.sum(jnp.where(pick1, logits, 0.0), axis=1, keepdims=True)

    sel2 = jnp.where(pick1, neg, sel)
    m2 = jnp.max(sel2, axis=1, keepdims=True)
    idx2 = jnp.min(jnp.where(sel2 >= m2, iota, E), axis=1, keepdims=True)
    pick2 = iota == idx2
    s2 = jnp.sum(jnp.where(pick2, logits, 0.0), axis=1, keepdims=True)

    g1 = 1.0 / (1.0 + jnp.exp(-s1))
    g2 = 1.0 / (1.0 + jnp.exp(-s2))
    denom = g1 + g2 + 1e-6
    wa_ref[...] = g1 / denom
    wb_ref[...] = g2 / denom

    oh = jnp.concatenate([pick1, pick2], axis=0).astype(jnp.int32)
    cum = oh
    sh = 1
    while sh < 2 * N:
        top = jnp.zeros((sh, E), jnp.int32)
        cum = cum + jnp.concatenate([top, cum[: 2 * N - sh]], axis=0)
        sh *= 2
    counts = cum[2 * N - 1 : 2 * N, :]
    nblk_e = (counts + (BM - 1)) // BM
    ii = lax.broadcasted_iota(jnp.int32, (E, E), 0)
    jj = lax.broadcasted_iota(jnp.int32, (E, E), 1)
    tri = (ii <= jj).astype(jnp.float32)
    blk_end = lax.dot_general(nblk_e.astype(jnp.float32), tri,
                              (((1,), (0,)), ((), ())),
                              preferred_element_type=jnp.float32).astype(jnp.int32)
    base_e = (blk_end - nblk_e) * BM
    rank = jnp.sum(cum * oh, axis=1, keepdims=True) - 1
    base_q = jnp.sum(oh * base_e, axis=1, keepdims=True)
    pos_ref[...] = rank + base_q

    gi = lax.broadcasted_iota(jnp.int32, (NBLK, E), 0)
    be = jnp.sum((gi >= blk_end).astype(jnp.int32), axis=1, keepdims=True)
    be_ref[...] = jnp.minimum(be, E - 1)
    nb_ref[...] = blk_end[:, E - 1 :]


def _router(x2d, gate_w, lb2d):
    col = lambda n: pl.BlockSpec((n, 1), lambda: (0, 0))
    return pl.pallas_call(
        _router_body,
        in_specs=[
            pl.BlockSpec((N, D), lambda: (0, 0)),
            pl.BlockSpec((E, D), lambda: (0, 0)),
            pl.BlockSpec((1, E), lambda: (0, 0)),
        ],
        out_specs=[col(N), col(N), col(2 * N), col(NBLK), col(1)],
        out_shape=[
            jax.ShapeDtypeStruct((N, 1), jnp.float32),
            jax.ShapeDtypeStruct((N, 1), jnp.float32),
            jax.ShapeDtypeStruct((2 * N, 1), jnp.int32),
            jax.ShapeDtypeStruct((NBLK, 1), jnp.int32),
            jax.ShapeDtypeStruct((1, 1), jnp.int32),
        ],
        interpret=_INTERPRET,
    )(x2d, gate_w, lb2d)


def _sc_dispatch(x2d, pos_flat):
    mesh = plsc.VectorSubcoreMesh(core_axis_name="c", subcore_axis_name="s",
                                  num_cores=NC, num_subcores=NS)

    @functools.partial(
        pl.kernel,
        out_type=jax.ShapeDtypeStruct((P, D), jnp.float32),
        mesh=mesh,
        scratch_types=[
            pltpu.VMEM((2, TPW), jnp.int32),
            pltpu.VMEM((TPW, D), jnp.float32),
            pltpu.SemaphoreType.DMA,
        ],
    )
    def k(xp_hbm, pos_hbm, out_hbm, idx_v, xbuf, sem):
        wid = lax.axis_index("s") * NC + lax.axis_index("c")
        pltpu.sync_copy(pos_hbm.at[pl.ds(wid * TPW, TPW)], idx_v.at[0])
        pltpu.sync_copy(pos_hbm.at[pl.ds(N + wid * TPW, TPW)], idx_v.at[1])
        pltpu.sync_copy(xp_hbm.at[pl.ds(wid * TPW, TPW)], xbuf)
        d0 = pltpu.async_copy(xbuf, out_hbm.at[idx_v.at[0]], sem)
        d1 = pltpu.async_copy(xbuf, out_hbm.at[idx_v.at[1]], sem)
        d0.wait()
        d1.wait()

    return k(x2d, pos_flat)


def _sc_gather(table, idx, n_rows, width):
    rows_per_w = n_rows // NW
    ch = rows_per_w
    while ch * width * 4 > 220 * 1024:
        ch //= 2
    n_ch = rows_per_w // ch
    mesh = plsc.VectorSubcoreMesh(core_axis_name="c", subcore_axis_name="s",
                                  num_cores=NC, num_subcores=NS)

    @functools.partial(
        pl.kernel,
        out_type=jax.ShapeDtypeStruct((n_rows, width), jnp.float32),
        mesh=mesh,
        scratch_types=[
            pltpu.VMEM((rows_per_w,), jnp.int32),
            pltpu.VMEM((ch, width), jnp.float32),
            pltpu.VMEM((ch, width), jnp.float32),
            pltpu.SemaphoreType.DMA,
            pltpu.SemaphoreType.DMA,
        ],
    )
    def k(table_hbm, idx_hbm, out_hbm, idx_v, buf0, buf1, sem0, sem1):
        wid = lax.axis_index("s") * NC + lax.axis_index("c")
        base = wid * rows_per_w
        pltpu.sync_copy(idx_hbm.at[pl.ds(base, rows_per_w)], idx_v)
        bufs = (buf0, buf1)
        sems = (sem0, sem1)
        descs = [None, None]
        for c in range(n_ch):
            descs[c % 2] = pltpu.async_copy(
                table_hbm.at[idx_v.at[pl.ds(c * ch, ch)]], bufs[c % 2], sems[c % 2])
            if c > 0:
                descs[(c - 1) % 2].wait()
                pltpu.sync_copy(bufs[(c - 1) % 2],
                                out_hbm.at[pl.ds(base + (c - 1) * ch, ch)])
        descs[(n_ch - 1) % 2].wait()
        pltpu.sync_copy(bufs[(n_ch - 1) % 2],
                        out_hbm.at[pl.ds(base + (n_ch - 1) * ch, ch)])

    return k(table, idx)


def _pack(a16):
    r, d = a16.shape
    return lax.bitcast_convert_type(a16.reshape(r, d // 2, 2), jnp.float32)


def _unpack(ap):
    r, dp = ap.shape
    return lax.bitcast_convert_type(ap, jnp.bfloat16).reshape(r, dp * 2)


def _ffn_body(be_ref, xg_ref, w1_ref, w2_ref, y_ref):
    xb = xg_ref[...]
    h = lax.dot_general(xb, w1_ref[0], (((1,), (1,)), ((), ())),
                        preferred_element_type=jnp.float32)
    h = jnp.square(jnp.maximum(h, 0.0))
    y_ref[...] = lax.dot_general(h, w2_ref[0], (((1,), (1,)), ((), ())),
                                 preferred_element_type=jnp.float32)


def _ffn(xg, w1, w2, block_expert, nblk):
    grid = (nblk,) if _DYN_GRID else (NBLK,)
    grid_spec = pltpu.PrefetchScalarGridSpec(
        num_scalar_prefetch=1,
        grid=grid,
        in_specs=[
            pl.BlockSpec((BM, D), lambda g, be: (g, 0)),
            pl.BlockSpec((1, F, D), lambda g, be: (be[g], 0, 0)),
            pl.BlockSpec((1, D, F), lambda g, be: (be[g], 0, 0)),
        ],
        out_specs=pl.BlockSpec((BM, D), lambda g, be: (g, 0)),
    )
    return pl.pallas_call(
        _ffn_body,
        grid_spec=grid_spec,
        out_shape=jax.ShapeDtypeStruct((P, D), jnp.float32),
        interpret=_INTERPRET,
    )(block_expert, xg, w1, w2)


_SB = 256


def _shared_body(x_ref, sfc_ref, spr_ref, out_ref):
    hs = lax.dot_general(x_ref[...], sfc_ref[...], (((1,), (1,)), ((), ())),
                         preferred_element_type=jnp.float32)
    hs = jnp.square(jnp.maximum(hs, 0.0))
    out_ref[...] = lax.dot_general(hs, spr_ref[...], (((1,), (1,)), ((), ())),
                                   preferred_element_type=jnp.float32)


def _shared(x2d, sfc, spr):
    return pl.pallas_call(
        _shared_body,
        grid=(N // _SB,),
        in_specs=[
            pl.BlockSpec((_SB, D), lambda g: (g, 0)),
            pl.BlockSpec((F, D), lambda g: (0, 0)),
            pl.BlockSpec((D, F), lambda g: (0, 0)),
        ],
        out_specs=pl.BlockSpec((_SB, D), lambda g: (g, 0)),
        out_shape=jax.ShapeDtypeStruct((N, D), jnp.float32),
        interpret=_INTERPRET,
    )(x2d, sfc, spr)


_CB = 256


def _combine_body(sh_ref, a_ref, b_ref, wa_ref, wb_ref, out_ref):
    out_ref[...] = (sh_ref[...]
                    + wa_ref[...] * a_ref[...]
                    + wb_ref[...] * b_ref[...])


def _combine(sh, ab, wa, wb):
    blk = lambda g: (g, 0)
    return pl.pallas_call(
        _combine_body,
        grid=(N // _CB,),
        in_specs=[
            pl.BlockSpec((_CB, D), blk),
            pl.BlockSpec((_CB, D), blk),
            pl.BlockSpec((_CB, D), lambda g: (g + N // _CB, 0)),
            pl.BlockSpec((_CB, 1), blk),
            pl.BlockSpec((_CB, 1), blk),
        ],
        out_specs=pl.BlockSpec((_CB, D), blk),
        out_shape=jax.ShapeDtypeStruct((N, D), jnp.float32),
        interpret=_INTERPRET,
    )(sh, ab, ab, wa, wb)


def kernel(x, gate_w, lb_bias, w1, w2, shared_fc, shared_proj):
    bsz, t, d = x.shape
    x2d = x.reshape(t * bsz, d)
    wa, wb, pos, be_col, nb = _router(x2d, gate_w, lb_bias.reshape(1, E))
    pos_flat = pos.reshape(2 * N)
    block_expert = be_col.reshape(NBLK)
    sh = _shared(x2d, shared_fc, shared_proj)
    if _USE_SC:
        xg = _sc_dispatch(x2d, pos_flat)
    else:
        slot_token = jnp.zeros((P,), jnp.int32).at[pos_flat].set(
            jnp.tile(jnp.arange(N, dtype=jnp.int32), 2))
        xg = jnp.take(x2d, slot_token, axis=0)
    y = _ffn(xg, w1, w2, block_expert, nb.reshape(())[()])
    if _USE_SC:
        ab = _sc_gather(y, pos_flat, 2 * N, D)
    else:
        ab = jnp.take(y, pos_flat, axis=0)
    out = _combine(sh, ab, wa, wb)
    return out.reshape(bsz, t, d)


# speedup vs baseline: 4.6673x; 1.1982x over previous
import functools

import jax
import jax.numpy as jnp
from jax import lax
from jax.experimental import pallas as pl
from jax.experimental.pallas import tpu as pltpu
from jax.experimental.pallas import tpu_sc as plsc

_INTERPRET = False
_USE_SC = True
_DYN_GRID = True

E = 8
D = 1024
F = 1024
N = 2048
DP = D // 2
BM = 256
NBLK = 23
P = NBLK * BM
NC, NS = 2, 16
NW = NC * NS
TPW = N // NW


def _router_body(x_ref, gw_ref, lb_ref,
                 wa_ref, wb_ref, pos_ref, be_ref, nb_ref):
    xb = x_ref[...]
    logits = lax.dot_general(xb, gw_ref[...], (((1,), (1,)), ((), ())),
                             preferred_element_type=jnp.float32)
    sel = logits + lb_ref[...]
    iota = lax.broadcasted_iota(jnp.int32, sel.shape, 1)
    neg = jnp.float32(-1e30)

    m1 = jnp.max(sel, axis=1, keepdims=True)
    idx1 = jnp.min(jnp.where(sel >= m1, iota, E), axis=1, keepdims=True)
    pick1 = iota == idx1
    s1 = jnp.sum(jnp.where(pick1, logits, 0.0), axis=1, keepdims=True)

    sel2 = jnp.where(pick1, neg, sel)
    m2 = jnp.max(sel2, axis=1, keepdims=True)
    idx2 = jnp.min(jnp.where(sel2 >= m2, iota, E), axis=1, keepdims=True)
    pick2 = iota == idx2
    s2 = jnp.sum(jnp.where(pick2, logits, 0.0), axis=1, keepdims=True)

    g1 = 1.0 / (1.0 + jnp.exp(-s1))
    g2 = 1.0 / (1.0 + jnp.exp(-s2))
    denom = g1 + g2 + 1e-6
    wa_ref[...] = g1 / denom
    wb_ref[...] = g2 / denom

    oh = jnp.concatenate([pick1, pick2], axis=0).astype(jnp.int32)
    cum = oh
    sh = 1
    while sh < 2 * N:
        top = jnp.zeros((sh, E), jnp.int32)
        cum = cum + jnp.concatenate([top, cum[: 2 * N - sh]], axis=0)
        sh *= 2
    counts = cum[2 * N - 1 : 2 * N, :]
    nblk_e = (counts + (BM - 1)) // BM
    ii = lax.broadcasted_iota(jnp.int32, (E, E), 0)
    jj = lax.broadcasted_iota(jnp.int32, (E, E), 1)
    tri = (ii <= jj).astype(jnp.float32)
    blk_end = lax.dot_general(nblk_e.astype(jnp.float32), tri,
                              (((1,), (0,)), ((), ())),
                              preferred_element_type=jnp.float32).astype(jnp.int32)
    base_e = (blk_end - nblk_e) * BM
    rank = jnp.sum(cum * oh, axis=1, keepdims=True) - 1
    base_q = jnp.sum(oh * base_e, axis=1, keepdims=True)
    pos_ref[...] = rank + base_q

    gi = lax.broadcasted_iota(jnp.int32, (NBLK, E), 0)
    be = jnp.sum((gi >= blk_end).astype(jnp.int32), axis=1, keepdims=True)
    be_ref[...] = jnp.minimum(be, E - 1)
    nb_ref[...] = blk_end[:, E - 1 :]


def _router(x2d, gate_w, lb2d):
    col = lambda n: pl.BlockSpec((n, 1), lambda: (0, 0))
    return pl.pallas_call(
        _router_body,
        in_specs=[
            pl.BlockSpec((N, D), lambda: (0, 0)),
            pl.BlockSpec((E, D), lambda: (0, 0)),
            pl.BlockSpec((1, E), lambda: (0, 0)),
        ],
        out_specs=[col(N), col(N), col(2 * N), col(NBLK), col(1)],
        out_shape=[
            jax.ShapeDtypeStruct((N, 1), jnp.float32),
            jax.ShapeDtypeStruct((N, 1), jnp.float32),
            jax.ShapeDtypeStruct((2 * N, 1), jnp.int32),
            jax.ShapeDtypeStruct((NBLK, 1), jnp.int32),
            jax.ShapeDtypeStruct((1, 1), jnp.int32),
        ],
        interpret=_INTERPRET,
    )(x2d, gate_w, lb2d)


def _sc_dispatch(x2d, pos_flat):
    mesh = plsc.VectorSubcoreMesh(core_axis_name="c", subcore_axis_name="s",
                                  num_cores=NC, num_subcores=NS)

    @functools.partial(
        pl.kernel,
        out_type=jax.ShapeDtypeStruct((P, D), jnp.float32),
        mesh=mesh,
        scratch_types=[
            pltpu.VMEM((2, TPW), jnp.int32),
            pltpu.VMEM((TPW, D), jnp.float32),
            pltpu.SemaphoreType.DMA,
        ],
    )
    def k(xp_hbm, pos_hbm, out_hbm, idx_v, xbuf, sem):
        wid = lax.axis_index("s") * NC + lax.axis_index("c")
        pltpu.sync_copy(pos_hbm.at[pl.ds(wid * TPW, TPW)], idx_v.at[0])
        pltpu.sync_copy(pos_hbm.at[pl.ds(N + wid * TPW, TPW)], idx_v.at[1])
        pltpu.sync_copy(xp_hbm.at[pl.ds(wid * TPW, TPW)], xbuf)
        d0 = pltpu.async_copy(xbuf, out_hbm.at[idx_v.at[0]], sem)
        d1 = pltpu.async_copy(xbuf, out_hbm.at[idx_v.at[1]], sem)
        d0.wait()
        d1.wait()

    return k(x2d, pos_flat)


def _sc_gather(table, idx, n_rows, width):
    rows_per_w = n_rows // NW
    ch = rows_per_w
    while ch * width * 4 > 220 * 1024:
        ch //= 2
    n_ch = rows_per_w // ch
    mesh = plsc.VectorSubcoreMesh(core_axis_name="c", subcore_axis_name="s",
                                  num_cores=NC, num_subcores=NS)

    @functools.partial(
        pl.kernel,
        out_type=jax.ShapeDtypeStruct((n_rows, width), jnp.float32),
        mesh=mesh,
        scratch_types=[
            pltpu.VMEM((rows_per_w,), jnp.int32),
            pltpu.VMEM((ch, width), jnp.float32),
            pltpu.VMEM((ch, width), jnp.float32),
            pltpu.SemaphoreType.DMA,
            pltpu.SemaphoreType.DMA,
        ],
    )
    def k(table_hbm, idx_hbm, out_hbm, idx_v, buf0, buf1, sem0, sem1):
        wid = lax.axis_index("s") * NC + lax.axis_index("c")
        base = wid * rows_per_w
        pltpu.sync_copy(idx_hbm.at[pl.ds(base, rows_per_w)], idx_v)
        bufs = (buf0, buf1)
        sems = (sem0, sem1)
        descs = [None, None]
        for c in range(n_ch):
            descs[c % 2] = pltpu.async_copy(
                table_hbm.at[idx_v.at[pl.ds(c * ch, ch)]], bufs[c % 2], sems[c % 2])
            if c > 0:
                descs[(c - 1) % 2].wait()
                pltpu.sync_copy(bufs[(c - 1) % 2],
                                out_hbm.at[pl.ds(base + (c - 1) * ch, ch)])
        descs[(n_ch - 1) % 2].wait()
        pltpu.sync_copy(bufs[(n_ch - 1) % 2],
                        out_hbm.at[pl.ds(base + (n_ch - 1) * ch, ch)])

    return k(table, idx)


def _pack(a16):
    r, d = a16.shape
    return lax.bitcast_convert_type(a16.reshape(r, d // 2, 2), jnp.float32)


def _unpack(ap):
    r, dp = ap.shape
    return lax.bitcast_convert_type(ap, jnp.bfloat16).reshape(r, dp * 2)


def _ffn_body(be_ref, xg_ref, w1_ref, w2_ref, y_ref):
    xb = xg_ref[...]
    h = lax.dot_general(xb, w1_ref[0], (((1,), (1,)), ((), ())),
                        preferred_element_type=jnp.float32)
    h = jnp.square(jnp.maximum(h, 0.0))
    y_ref[...] = lax.dot_general(h, w2_ref[0], (((1,), (1,)), ((), ())),
                                 preferred_element_type=jnp.float32)


def _ffn(xg, w1, w2, block_expert, nblk):
    grid = (nblk,) if _DYN_GRID else (NBLK,)
    grid_spec = pltpu.PrefetchScalarGridSpec(
        num_scalar_prefetch=1,
        grid=grid,
        in_specs=[
            pl.BlockSpec((BM, D), lambda g, be: (g, 0)),
            pl.BlockSpec((1, F, D), lambda g, be: (be[g], 0, 0)),
            pl.BlockSpec((1, D, F), lambda g, be: (be[g], 0, 0)),
        ],
        out_specs=pl.BlockSpec((BM, D), lambda g, be: (g, 0)),
    )
    return pl.pallas_call(
        _ffn_body,
        grid_spec=grid_spec,
        out_shape=jax.ShapeDtypeStruct((P, D), jnp.float32),
        interpret=_INTERPRET,
    )(block_expert, xg, w1, w2)


_SB = 256


def _shared_body(x_ref, sfc_ref, spr_ref, out_ref):
    hs = lax.dot_general(x_ref[...], sfc_ref[...], (((1,), (1,)), ((), ())),
                         preferred_element_type=jnp.float32)
    hs = jnp.square(jnp.maximum(hs, 0.0))
    out_ref[...] = lax.dot_general(hs, spr_ref[...], (((1,), (1,)), ((), ())),
                                   preferred_element_type=jnp.float32)


def _shared(x2d, sfc, spr):
    return pl.pallas_call(
        _shared_body,
        grid=(N // _SB,),
        in_specs=[
            pl.BlockSpec((_SB, D), lambda g: (g, 0)),
            pl.BlockSpec((F, D), lambda g: (0, 0)),
            pl.BlockSpec((D, F), lambda g: (0, 0)),
        ],
        out_specs=pl.BlockSpec((_SB, D), lambda g: (g, 0)),
        out_shape=jax.ShapeDtypeStruct((N, D), jnp.float32),
        interpret=_INTERPRET,
    )(x2d, sfc, spr)


_CB = 256


def _combine_body(sh_ref, a_ref, b_ref, wa_ref, wb_ref, out_ref):
    out_ref[...] = (sh_ref[...]
                    + wa_ref[...] * a_ref[...]
                    + wb_ref[...] * b_ref[...])


def _combine(sh, ab, wa, wb):
    blk = lambda g: (g, 0)
    return pl.pallas_call(
        _combine_body,
        grid=(N // _CB,),
        in_specs=[
            pl.BlockSpec((_CB, D), blk),
            pl.BlockSpec((_CB, D), blk),
            pl.BlockSpec((_CB, D), lambda g: (g + N // _CB, 0)),
            pl.BlockSpec((_CB, 1), blk),
            pl.BlockSpec((_CB, 1), blk),
        ],
        out_specs=pl.BlockSpec((_CB, D), blk),
        out_shape=jax.ShapeDtypeStruct((N, D), jnp.float32),
        interpret=_INTERPRET,
    )(sh, ab, ab, wa, wb)


def kernel(x, gate_w, lb_bias, w1, w2, shared_fc, shared_proj):
    bsz, t, d = x.shape
    x2d = x.reshape(t * bsz, d)
    wa, wb, pos, be_col, nb = _router(x2d, gate_w, lb_bias.reshape(1, E))
    pos_flat = pos.reshape(2 * N)
    block_expert = be_col.reshape(NBLK)
    sh = _shared(x2d, shared_fc, shared_proj)
    if _USE_SC:
        xg = _sc_dispatch(x2d, pos_flat)
    else:
        slot_token = jnp.zeros((P,), jnp.int32).at[pos_flat].set(
            jnp.tile(jnp.arange(N, dtype=jnp.int32), 2))
        xg = jnp.take(x2d, slot_token, axis=0)
    y = _ffn(xg, w1, w2, block_expert, nb.reshape(())[()])
    if _USE_SC:
        ab = _sc_gather(y, pos_flat, 2 * N, D)
    else:
        ab = jnp.take(y, pos_flat, axis=0)
    out = _combine(sh, ab, wa, wb)
    return out.reshape(bsz, t, d)


# speedup vs baseline: 5.0442x; 1.0808x over previous
import functools

import jax
import jax.numpy as jnp
from jax import lax
from jax.experimental import pallas as pl
from jax.experimental.pallas import tpu as pltpu
from jax.experimental.pallas import tpu_sc as plsc

_INTERPRET = False
_USE_SC = True
_DYN_GRID = True

E = 8
D = 1024
F = 1024
N = 2048
DP = D // 2
BM = 256
NBLK = 23
P = NBLK * BM
NC, NS = 2, 16
NW = NC * NS
TPW = N // NW


def _router_body(x_ref, gw_ref, lb_ref,
                 wa_ref, wb_ref, pos_ref, meta_ref, nb_ref):
    xb = x_ref[...]
    logits = lax.dot_general(xb, gw_ref[...], (((1,), (1,)), ((), ())),
                             preferred_element_type=jnp.float32)
    sel = logits + lb_ref[...]
    iota = lax.broadcasted_iota(jnp.int32, sel.shape, 1)
    neg = jnp.float32(-1e30)

    m1 = jnp.max(sel, axis=1, keepdims=True)
    idx1 = jnp.min(jnp.where(sel >= m1, iota, E), axis=1, keepdims=True)
    pick1 = iota == idx1
    s1 = jnp.sum(jnp.where(pick1, logits, 0.0), axis=1, keepdims=True)

    sel2 = jnp.where(pick1, neg, sel)
    m2 = jnp.max(sel2, axis=1, keepdims=True)
    idx2 = jnp.min(jnp.where(sel2 >= m2, iota, E), axis=1, keepdims=True)
    pick2 = iota == idx2
    s2 = jnp.sum(jnp.where(pick2, logits, 0.0), axis=1, keepdims=True)

    g1 = 1.0 / (1.0 + jnp.exp(-s1))
    g2 = 1.0 / (1.0 + jnp.exp(-s2))
    denom = g1 + g2 + 1e-6
    wa_ref[...] = g1 / denom
    wb_ref[...] = g2 / denom

    oh = jnp.concatenate([pick1, pick2], axis=0).astype(jnp.int32)
    cum = oh
    sh = 1
    while sh < 2 * N:
        top = jnp.zeros((sh, E), jnp.int32)
        cum = cum + jnp.concatenate([top, cum[: 2 * N - sh]], axis=0)
        sh *= 2
    counts = cum[2 * N - 1 : 2 * N, :]
    nblk_e = (counts + (BM - 1)) // BM
    ii = lax.broadcasted_iota(jnp.int32, (E, E), 0)
    jj = lax.broadcasted_iota(jnp.int32, (E, E), 1)
    tri = (ii <= jj).astype(jnp.float32)
    blk_end = lax.dot_general(nblk_e.astype(jnp.float32), tri,
                              (((1,), (0,)), ((), ())),
                              preferred_element_type=jnp.float32).astype(jnp.int32)
    base_e = (blk_end - nblk_e) * BM
    rank = jnp.sum(cum * oh, axis=1, keepdims=True) - 1
    base_q = jnp.sum(oh * base_e, axis=1, keepdims=True)
    pos_ref[...] = rank + base_q

    gi = lax.broadcasted_iota(jnp.int32, (NBLK, E), 0)
    ei = lax.broadcasted_iota(jnp.int32, (NBLK, E), 1)
    be = jnp.minimum(
        jnp.sum((gi >= blk_end).astype(jnp.int32), axis=1, keepdims=True), E - 1)
    ohg = ei == be
    nonempty = counts > 0
    blk_start_row = blk_end - nblk_e
    rs = jnp.sum((ohg & (gi == blk_start_row)).astype(jnp.int32),
                 axis=1, keepdims=True)
    r = jnp.sum(((ei <= be) & nonempty).astype(jnp.int32),
                axis=1, keepdims=True) - 1
    rp = r & 1
    ne = jnp.min(jnp.where((ei > be) & nonempty, ei, E),
                 axis=1, keepdims=True)
    hn = (ne < E).astype(jnp.int32)
    zero = jnp.zeros((NBLK, 1), jnp.int32)
    meta_ref[...] = jnp.concatenate(
        [be, rs, rp, jnp.minimum(ne, E - 1), hn, zero, zero, zero],
        axis=1)
    nb_ref[...] = blk_end[:, E - 1 :]


def _router(x2d, gate_w, lb2d):
    col = lambda n: pl.BlockSpec((n, 1), lambda: (0, 0))
    return pl.pallas_call(
        _router_body,
        in_specs=[
            pl.BlockSpec((N, D), lambda: (0, 0)),
            pl.BlockSpec((E, D), lambda: (0, 0)),
            pl.BlockSpec((1, E), lambda: (0, 0)),
        ],
        out_specs=[col(N), col(N), col(2 * N),
                   pl.BlockSpec((NBLK, E), lambda: (0, 0)), col(1)],
        out_shape=[
            jax.ShapeDtypeStruct((N, 1), jnp.float32),
            jax.ShapeDtypeStruct((N, 1), jnp.float32),
            jax.ShapeDtypeStruct((2 * N, 1), jnp.int32),
            jax.ShapeDtypeStruct((NBLK, E), jnp.int32),
            jax.ShapeDtypeStruct((1, 1), jnp.int32),
        ],
        interpret=_INTERPRET,
    )(x2d, gate_w, lb2d)


def _sc_dispatch(x2d, pos_flat):
    mesh = plsc.VectorSubcoreMesh(core_axis_name="c", subcore_axis_name="s",
                                  num_cores=NC, num_subcores=NS)

    @functools.partial(
        pl.kernel,
        out_type=jax.ShapeDtypeStruct((P, D), jnp.float32),
        mesh=mesh,
        scratch_types=[
            pltpu.VMEM((2, TPW), jnp.int32),
            pltpu.VMEM((TPW, D), jnp.float32),
            pltpu.SemaphoreType.DMA,
        ],
    )
    def k(xp_hbm, pos_hbm, out_hbm, idx_v, xbuf, sem):
        wid = lax.axis_index("s") * NC + lax.axis_index("c")
        pltpu.sync_copy(pos_hbm.at[pl.ds(wid * TPW, TPW)], idx_v.at[0])
        pltpu.sync_copy(pos_hbm.at[pl.ds(N + wid * TPW, TPW)], idx_v.at[1])
        pltpu.sync_copy(xp_hbm.at[pl.ds(wid * TPW, TPW)], xbuf)
        d0 = pltpu.async_copy(xbuf, out_hbm.at[idx_v.at[0]], sem)
        d1 = pltpu.async_copy(xbuf, out_hbm.at[idx_v.at[1]], sem)
        d0.wait()
        d1.wait()

    return k(x2d, pos_flat)


def _sc_gather(table, idx, n_rows, width):
    rows_per_w = n_rows // NW
    ch = rows_per_w
    while ch * width * 4 > 220 * 1024:
        ch //= 2
    n_ch = rows_per_w // ch
    mesh = plsc.VectorSubcoreMesh(core_axis_name="c", subcore_axis_name="s",
                                  num_cores=NC, num_subcores=NS)

    @functools.partial(
        pl.kernel,
        out_type=jax.ShapeDtypeStruct((n_rows, width), jnp.float32),
        mesh=mesh,
        scratch_types=[
            pltpu.VMEM((rows_per_w,), jnp.int32),
            pltpu.VMEM((ch, width), jnp.float32),
            pltpu.VMEM((ch, width), jnp.float32),
            pltpu.SemaphoreType.DMA,
            pltpu.SemaphoreType.DMA,
        ],
    )
    def k(table_hbm, idx_hbm, out_hbm, idx_v, buf0, buf1, sem0, sem1):
        wid = lax.axis_index("s") * NC + lax.axis_index("c")
        base = wid * rows_per_w
        pltpu.sync_copy(idx_hbm.at[pl.ds(base, rows_per_w)], idx_v)
        bufs = (buf0, buf1)
        sems = (sem0, sem1)
        descs = [None, None]
        for c in range(n_ch):
            descs[c % 2] = pltpu.async_copy(
                table_hbm.at[idx_v.at[pl.ds(c * ch, ch)]], bufs[c % 2], sems[c % 2])
            if c > 0:
                descs[(c - 1) % 2].wait()
                pltpu.sync_copy(bufs[(c - 1) % 2],
                                out_hbm.at[pl.ds(base + (c - 1) * ch, ch)])
        descs[(n_ch - 1) % 2].wait()
        pltpu.sync_copy(bufs[(n_ch - 1) % 2],
                        out_hbm.at[pl.ds(base + (n_ch - 1) * ch, ch)])

    return k(table, idx)


def _pack(a16):
    r, d = a16.shape
    return lax.bitcast_convert_type(a16.reshape(r, d // 2, 2), jnp.float32)


def _unpack(ap):
    r, dp = ap.shape
    return lax.bitcast_convert_type(ap, jnp.bfloat16).reshape(r, dp * 2)


def _ffn_body(meta_ref, xg_ref, w1_hbm, w2_hbm, y_ref,
              wbuf1, wbuf2, sem1, sem2):
    g = pl.program_id(0)
    par = meta_ref[g, 2]

    @pl.when(g == 0)
    def _():
        pltpu.make_async_copy(w1_hbm.at[meta_ref[0, 0]], wbuf1.at[0], sem1).start()
        pltpu.make_async_copy(w2_hbm.at[meta_ref[0, 0]], wbuf2.at[0], sem2).start()

    @pl.when(meta_ref[g, 1] == 1)
    def _():
        pltpu.make_async_copy(w1_hbm.at[meta_ref[g, 0]], wbuf1.at[par], sem1).wait()
        pltpu.make_async_copy(w2_hbm.at[meta_ref[g, 0]], wbuf2.at[par], sem2).wait()

    @pl.when((meta_ref[g, 1] == 1) & (meta_ref[g, 4] == 1))
    def _():
        nxt = meta_ref[g, 3]
        pltpu.make_async_copy(w1_hbm.at[nxt], wbuf1.at[1 - par], sem1).start()
        pltpu.make_async_copy(w2_hbm.at[nxt], wbuf2.at[1 - par], sem2).start()

    xb = xg_ref[...]
    h = lax.dot_general(xb, wbuf1[par], (((1,), (1,)), ((), ())),
                        preferred_element_type=jnp.float32)
    h = jnp.square(jnp.maximum(h, 0.0))
    y_ref[...] = lax.dot_general(h, wbuf2[par], (((1,), (1,)), ((), ())),
                                 preferred_element_type=jnp.float32)


def _ffn(xg, w1, w2, meta, nblk):
    grid = (nblk,) if _DYN_GRID else (NBLK,)
    grid_spec = pltpu.PrefetchScalarGridSpec(
        num_scalar_prefetch=1,
        grid=grid,
        in_specs=[
            pl.BlockSpec((BM, D), lambda g, m: (g, 0)),
            pl.BlockSpec(memory_space=pl.ANY),
            pl.BlockSpec(memory_space=pl.ANY),
        ],
        out_specs=pl.BlockSpec((BM, D), lambda g, m: (g, 0)),
        scratch_shapes=[
            pltpu.VMEM((2, F, D), jnp.float32),
            pltpu.VMEM((2, D, F), jnp.float32),
            pltpu.SemaphoreType.DMA,
            pltpu.SemaphoreType.DMA,
        ],
    )
    return pl.pallas_call(
        _ffn_body,
        grid_spec=grid_spec,
        out_shape=jax.ShapeDtypeStruct((P, D), jnp.float32),
        interpret=_INTERPRET,
    )(meta, xg, w1, w2)


_SB = 256


def _shared_body(x_ref, sfc_ref, spr_ref, out_ref):
    hs = lax.dot_general(x_ref[...], sfc_ref[...], (((1,), (1,)), ((), ())),
                         preferred_element_type=jnp.float32)
    hs = jnp.square(jnp.maximum(hs, 0.0))
    out_ref[...] = lax.dot_general(hs, spr_ref[...], (((1,), (1,)), ((), ())),
                                   preferred_element_type=jnp.float32)


def _shared(x2d, sfc, spr):
    return pl.pallas_call(
        _shared_body,
        grid=(N // _SB,),
        in_specs=[
            pl.BlockSpec((_SB, D), lambda g: (g, 0)),
            pl.BlockSpec((F, D), lambda g: (0, 0)),
            pl.BlockSpec((D, F), lambda g: (0, 0)),
        ],
        out_specs=pl.BlockSpec((_SB, D), lambda g: (g, 0)),
        out_shape=jax.ShapeDtypeStruct((N, D), jnp.float32),
        interpret=_INTERPRET,
    )(x2d, sfc, spr)


_CB = 256


def _combine_body(sh_ref, a_ref, b_ref, wa_ref, wb_ref, out_ref):
    out_ref[...] = (sh_ref[...]
                    + wa_ref[...] * a_ref[...]
                    + wb_ref[...] * b_ref[...])


def _combine(sh, ab, wa, wb):
    blk = lambda g: (g, 0)
    return pl.pallas_call(
        _combine_body,
        grid=(N // _CB,),
        in_specs=[
            pl.BlockSpec((_CB, D), blk),
            pl.BlockSpec((_CB, D), blk),
            pl.BlockSpec((_CB, D), lambda g: (g + N // _CB, 0)),
            pl.BlockSpec((_CB, 1), blk),
            pl.BlockSpec((_CB, 1), blk),
        ],
        out_specs=pl.BlockSpec((_CB, D), blk),
        out_shape=jax.ShapeDtypeStruct((N, D), jnp.float32),
        interpret=_INTERPRET,
    )(sh, ab, ab, wa, wb)


def kernel(x, gate_w, lb_bias, w1, w2, shared_fc, shared_proj):
    bsz, t, d = x.shape
    x2d = x.reshape(t * bsz, d)
    wa, wb, pos, meta, nb = _router(x2d, gate_w, lb_bias.reshape(1, E))
    pos_flat = pos.reshape(2 * N)
    sh = _shared(x2d, shared_fc, shared_proj)
    if _USE_SC:
        xg = _sc_dispatch(x2d, pos_flat)
    else:
        slot_token = jnp.zeros((P,), jnp.int32).at[pos_flat].set(
            jnp.tile(jnp.arange(N, dtype=jnp.int32), 2))
        xg = jnp.take(x2d, slot_token, axis=0)
    y = _ffn(xg, w1, w2, meta, nb.reshape(())[()])
    if _USE_SC:
        ab = _sc_gather(y, pos_flat, 2 * N, D)
    else:
        ab = jnp.take(y, pos_flat, axis=0)
    out = _combine(sh, ab, wa, wb)
    return out.reshape(bsz, t, d)


# speedup vs baseline: 5.1356x; 1.0181x over previous
import functools

import jax
import jax.numpy as jnp
from jax import lax
from jax.experimental import pallas as pl
from jax.experimental.pallas import tpu as pltpu
from jax.experimental.pallas import tpu_sc as plsc

E = 8
D = 1024
F = 1024
N = 2048
BM = 256
NBLK = 23
P = NBLK * BM
NC, NS = 2, 16
NW = NC * NS
TPW = N // NW


def _router_body(x_ref, gw_ref, lb_ref,
                 wa_ref, wb_ref, pos_ref, meta_ref, nb_ref):
    xb = x_ref[...]
    logits = lax.dot_general(xb, gw_ref[...], (((1,), (1,)), ((), ())),
                             preferred_element_type=jnp.float32)
    sel = logits + lb_ref[...]
    iota = lax.broadcasted_iota(jnp.int32, sel.shape, 1)
    neg = jnp.float32(-1e30)

    m1 = jnp.max(sel, axis=1, keepdims=True)
    idx1 = jnp.min(jnp.where(sel >= m1, iota, E), axis=1, keepdims=True)
    pick1 = iota == idx1
    s1 = jnp.sum(jnp.where(pick1, logits, 0.0), axis=1, keepdims=True)

    sel2 = jnp.where(pick1, neg, sel)
    m2 = jnp.max(sel2, axis=1, keepdims=True)
    idx2 = jnp.min(jnp.where(sel2 >= m2, iota, E), axis=1, keepdims=True)
    pick2 = iota == idx2
    s2 = jnp.sum(jnp.where(pick2, logits, 0.0), axis=1, keepdims=True)

    g1 = 1.0 / (1.0 + jnp.exp(-s1))
    g2 = 1.0 / (1.0 + jnp.exp(-s2))
    denom = g1 + g2 + 1e-6
    wa_ref[...] = g1 / denom
    wb_ref[...] = g2 / denom

    oh = jnp.concatenate([pick1, pick2], axis=0).astype(jnp.int32)
    cum = oh
    sh = 1
    while sh < 2 * N:
        top = jnp.zeros((sh, E), jnp.int32)
        cum = cum + jnp.concatenate([top, cum[: 2 * N - sh]], axis=0)
        sh *= 2
    counts = cum[2 * N - 1 : 2 * N, :]
    nblk_e = (counts + (BM - 1)) // BM
    ii = lax.broadcasted_iota(jnp.int32, (E, E), 0)
    jj = lax.broadcasted_iota(jnp.int32, (E, E), 1)
    tri = (ii <= jj).astype(jnp.float32)
    blk_end = lax.dot_general(nblk_e.astype(jnp.float32), tri,
                              (((1,), (0,)), ((), ())),
                              preferred_element_type=jnp.float32).astype(jnp.int32)
    base_e = (blk_end - nblk_e) * BM
    rank = jnp.sum(cum * oh, axis=1, keepdims=True) - 1
    base_q = jnp.sum(oh * base_e, axis=1, keepdims=True)
    pos_ref[...] = rank + base_q

    gi = lax.broadcasted_iota(jnp.int32, (NBLK, E), 0)
    ei = lax.broadcasted_iota(jnp.int32, (NBLK, E), 1)
    be = jnp.minimum(
        jnp.sum((gi >= blk_end).astype(jnp.int32), axis=1, keepdims=True), E - 1)
    ohg = ei == be
    nonempty = counts > 0
    blk_start_row = blk_end - nblk_e
    rs = jnp.sum((ohg & (gi == blk_start_row)).astype(jnp.int32),
                 axis=1, keepdims=True)
    r = jnp.sum(((ei <= be) & nonempty).astype(jnp.int32),
                axis=1, keepdims=True) - 1
    rp = r & 1
    ne = jnp.min(jnp.where((ei > be) & nonempty, ei, E),
                 axis=1, keepdims=True)
    hn = (ne < E).astype(jnp.int32)
    zero = jnp.zeros((NBLK, 1), jnp.int32)
    meta_ref[...] = jnp.concatenate(
        [be, rs, rp, jnp.minimum(ne, E - 1), hn, zero, zero, zero],
        axis=1)
    nb_ref[...] = blk_end[:, E - 1 :]


def _router(x2d, gate_w, lb2d):
    col = lambda n: pl.BlockSpec((n, 1), lambda: (0, 0))
    return pl.pallas_call(
        _router_body,
        in_specs=[
            pl.BlockSpec((N, D), lambda: (0, 0)),
            pl.BlockSpec((E, D), lambda: (0, 0)),
            pl.BlockSpec((1, E), lambda: (0, 0)),
        ],
        out_specs=[col(N), col(N), col(2 * N),
                   pl.BlockSpec((NBLK, E), lambda: (0, 0)), col(1)],
        out_shape=[
            jax.ShapeDtypeStruct((N, 1), jnp.float32),
            jax.ShapeDtypeStruct((N, 1), jnp.float32),
            jax.ShapeDtypeStruct((2 * N, 1), jnp.int32),
            jax.ShapeDtypeStruct((NBLK, E), jnp.int32),
            jax.ShapeDtypeStruct((1, 1), jnp.int32),
        ],
    )(x2d, gate_w, lb2d)


def _sc_dispatch(x2d, pos_flat):
    mesh = plsc.VectorSubcoreMesh(core_axis_name="c", subcore_axis_name="s",
                                  num_cores=NC, num_subcores=NS)

    @functools.partial(
        pl.kernel,
        out_type=jax.ShapeDtypeStruct((P, D), jnp.float32),
        mesh=mesh,
        scratch_types=[
            pltpu.VMEM((2, TPW), jnp.int32),
            pltpu.VMEM((TPW, D), jnp.float32),
            pltpu.SemaphoreType.DMA,
        ],
    )
    def k(xp_hbm, pos_hbm, out_hbm, idx_v, xbuf, sem):
        wid = lax.axis_index("s") * NC + lax.axis_index("c")
        pltpu.sync_copy(pos_hbm.at[pl.ds(wid * TPW, TPW)], idx_v.at[0])
        pltpu.sync_copy(pos_hbm.at[pl.ds(N + wid * TPW, TPW)], idx_v.at[1])
        pltpu.sync_copy(xp_hbm.at[pl.ds(wid * TPW, TPW)], xbuf)
        d0 = pltpu.async_copy(xbuf, out_hbm.at[idx_v.at[0]], sem)
        d1 = pltpu.async_copy(xbuf, out_hbm.at[idx_v.at[1]], sem)
        d0.wait()
        d1.wait()

    return k(x2d, pos_flat)


def _sc_gather(table, idx, n_rows, width):
    rows_per_w = n_rows // NW
    ch = rows_per_w
    while ch * width * 4 > 220 * 1024:
        ch //= 2
    n_ch = rows_per_w // ch
    mesh = plsc.VectorSubcoreMesh(core_axis_name="c", subcore_axis_name="s",
                                  num_cores=NC, num_subcores=NS)

    @functools.partial(
        pl.kernel,
        out_type=jax.ShapeDtypeStruct((n_rows, width), jnp.float32),
        mesh=mesh,
        scratch_types=[
            pltpu.VMEM((rows_per_w,), jnp.int32),
            pltpu.VMEM((ch, width), jnp.float32),
            pltpu.VMEM((ch, width), jnp.float32),
            pltpu.SemaphoreType.DMA,
            pltpu.SemaphoreType.DMA,
        ],
    )
    def k(table_hbm, idx_hbm, out_hbm, idx_v, buf0, buf1, sem0, sem1):
        wid = lax.axis_index("s") * NC + lax.axis_index("c")
        base = wid * rows_per_w
        pltpu.sync_copy(idx_hbm.at[pl.ds(base, rows_per_w)], idx_v)
        bufs = (buf0, buf1)
        sems = (sem0, sem1)
        descs = [None, None]
        for c in range(n_ch):
            descs[c % 2] = pltpu.async_copy(
                table_hbm.at[idx_v.at[pl.ds(c * ch, ch)]], bufs[c % 2], sems[c % 2])
            if c > 0:
                descs[(c - 1) % 2].wait()
                pltpu.sync_copy(bufs[(c - 1) % 2],
                                out_hbm.at[pl.ds(base + (c - 1) * ch, ch)])
        descs[(n_ch - 1) % 2].wait()
        pltpu.sync_copy(bufs[(n_ch - 1) % 2],
                        out_hbm.at[pl.ds(base + (n_ch - 1) * ch, ch)])

    return k(table, idx)


def _ffn_body(meta_ref, xg_ref, w1_hbm, w2_hbm, y_ref,
              wbuf1, wbuf2, sem1, sem2):
    g = pl.program_id(0)
    par = meta_ref[g, 2]

    @pl.when(g == 0)
    def _():
        pltpu.make_async_copy(w1_hbm.at[meta_ref[0, 0]], wbuf1.at[0], sem1).start()
        pltpu.make_async_copy(w2_hbm.at[meta_ref[0, 0]], wbuf2.at[0], sem2).start()

    @pl.when(meta_ref[g, 1] == 1)
    def _():
        pltpu.make_async_copy(w1_hbm.at[meta_ref[g, 0]], wbuf1.at[par], sem1).wait()
        pltpu.make_async_copy(w2_hbm.at[meta_ref[g, 0]], wbuf2.at[par], sem2).wait()

    @pl.when((meta_ref[g, 1] == 1) & (meta_ref[g, 4] == 1))
    def _():
        nxt = meta_ref[g, 3]
        pltpu.make_async_copy(w1_hbm.at[nxt], wbuf1.at[1 - par], sem1).start()
        pltpu.make_async_copy(w2_hbm.at[nxt], wbuf2.at[1 - par], sem2).start()

    xb = xg_ref[...]
    h = lax.dot_general(xb, wbuf1[par], (((1,), (1,)), ((), ())),
                        preferred_element_type=jnp.float32)
    h = jnp.square(jnp.maximum(h, 0.0))
    y_ref[...] = lax.dot_general(h, wbuf2[par], (((1,), (1,)), ((), ())),
                                 preferred_element_type=jnp.float32)


def _ffn(xg, w1, w2, meta, nblk):
    grid = (nblk,)
    grid_spec = pltpu.PrefetchScalarGridSpec(
        num_scalar_prefetch=1,
        grid=grid,
        in_specs=[
            pl.BlockSpec((BM, D), lambda g, m: (g, 0)),
            pl.BlockSpec(memory_space=pl.ANY),
            pl.BlockSpec(memory_space=pl.ANY),
        ],
        out_specs=pl.BlockSpec((BM, D), lambda g, m: (g, 0)),
        scratch_shapes=[
            pltpu.VMEM((2, F, D), jnp.float32),
            pltpu.VMEM((2, D, F), jnp.float32),
            pltpu.SemaphoreType.DMA,
            pltpu.SemaphoreType.DMA,
        ],
    )
    return pl.pallas_call(
        _ffn_body,
        grid_spec=grid_spec,
        out_shape=jax.ShapeDtypeStruct((P, D), jnp.float32),
    )(meta, xg, w1, w2)


_SB = 256


def _shared_body(x_ref, sfc_ref, spr_ref, out_ref):
    hs = lax.dot_general(x_ref[...], sfc_ref[...], (((1,), (1,)), ((), ())),
                         preferred_element_type=jnp.float32)
    hs = jnp.square(jnp.maximum(hs, 0.0))
    out_ref[...] = lax.dot_general(hs, spr_ref[...], (((1,), (1,)), ((), ())),
                                   preferred_element_type=jnp.float32
                                   ).astype(jnp.bfloat16)


def _shared(x2d, sfc, spr):
    return pl.pallas_call(
        _shared_body,
        grid=(N // _SB,),
        in_specs=[
            pl.BlockSpec((_SB, D), lambda g: (g, 0)),
            pl.BlockSpec((F, D), lambda g: (0, 0)),
            pl.BlockSpec((D, F), lambda g: (0, 0)),
        ],
        out_specs=pl.BlockSpec((_SB, D), lambda g: (g, 0)),
        out_shape=jax.ShapeDtypeStruct((N, D), jnp.bfloat16),
    )(x2d, sfc, spr)


_CB = 256


def _combine_body(sh_ref, a_ref, b_ref, wa_ref, wb_ref, out_ref):
    out_ref[...] = (sh_ref[...].astype(jnp.float32)
                    + wa_ref[...] * a_ref[...]
                    + wb_ref[...] * b_ref[...])


def _combine(sh, ab, wa, wb):
    blk = lambda g: (g, 0)
    return pl.pallas_call(
        _combine_body,
        grid=(N // _CB,),
        in_specs=[
            pl.BlockSpec((_CB, D), blk),
            pl.BlockSpec((_CB, D), blk),
            pl.BlockSpec((_CB, D), lambda g: (g + N // _CB, 0)),
            pl.BlockSpec((_CB, 1), blk),
            pl.BlockSpec((_CB, 1), blk),
        ],
        out_specs=pl.BlockSpec((_CB, D), blk),
        out_shape=jax.ShapeDtypeStruct((N, D), jnp.float32),
    )(sh, ab, ab, wa, wb)


def kernel(x, gate_w, lb_bias, w1, w2, shared_fc, shared_proj):
    bsz, t, d = x.shape
    x2d = x.reshape(t * bsz, d)
    wa, wb, pos, meta, nb = _router(x2d, gate_w, lb_bias.reshape(1, E))
    pos_flat = pos.reshape(2 * N)
    sh = _shared(x2d, shared_fc, shared_proj)
    xg = _sc_dispatch(x2d, pos_flat)
    y = _ffn(xg, w1, w2, meta, nb.reshape(())[()])
    ab = _sc_gather(y, pos_flat, 2 * N, D)
    out = _combine(sh, ab, wa, wb)
    return out.reshape(bsz, t, d)
